# baseline (speedup 1.0000x reference)
import functools
import math

import jax
import jax.numpy as jnp
from jax import lax
from jax.experimental import pallas as pl
from jax.experimental.pallas import tpu as pltpu
from jax.experimental.pallas import tpu_sc as plsc

D = 64
SCALE = math.sqrt(D)
NC, NS = 2, 16
NW = NC * NS
G = 128
K = 8
ROWS_PER_GRP = G * K


def _emb_body(idx_hbm, table_hbm, out_hbm, idx_v, rows_v, sem):
    n_grp = idx_hbm.shape[0]
    grp_per_w = n_grp // NW
    iters = grp_per_w // K
    wid = lax.axis_index("s") * NC + lax.axis_index("c")

    def body(g, carry):
        grp0 = wid * grp_per_w + g * K
        pltpu.sync_copy(idx_hbm.at[pl.ds(grp0, K)], idx_v)
        copies = [
            pltpu.async_copy(table_hbm.at[idx_v.at[j]], rows_v.at[j], sem)
            for j in range(K)
        ]
        for c in copies:
            c.wait()

        def scale(i, c2):
            for j in range(K):
                for k in range(D // 16):
                    sl = pl.ds(k * 16, 16)
                    rows_v[j, i, sl] = rows_v[j, i, sl] * SCALE
            return c2

        lax.fori_loop(0, G, scale, None)
        pltpu.sync_copy(rows_v, out_hbm.at[pl.ds(grp0, K)])
        return carry

    lax.fori_loop(0, iters, body, None)


def kernel(x, lut):
    B = x.shape[0] * x.shape[1]
    idx = x.reshape(B // G, G)

    mesh = plsc.VectorSubcoreMesh(
        core_axis_name="c", subcore_axis_name="s",
        num_cores=NC, num_subcores=NS,
    )
    emb = pl.kernel(
        _emb_body,
        out_type=jax.ShapeDtypeStruct((B // G, G, D), jnp.float32),
        mesh=mesh,
        scratch_types=[
            pltpu.VMEM((K, G), jnp.int32),
            pltpu.VMEM((K, G, D), jnp.float32),
            pltpu.SemaphoreType.DMA,
        ],
        compiler_params=pltpu.CompilerParams(use_tc_tiling_on_sc=False),
    )
    out = emb(idx, lut)
    return out.reshape(x.shape[0], x.shape[1], D)


# speedup vs baseline: 1.0809x; 1.0809x over previous
import math

import jax
import jax.numpy as jnp
from jax import lax
from jax.experimental import pallas as pl
from jax.experimental.pallas import tpu as pltpu
from jax.experimental.pallas import tpu_sc as plsc

D = 64
SCALE = math.sqrt(D)
NC, NS = 2, 16
NW = NC * NS
G = 128
K = 4


def _emb_body(idx_hbm, table_hbm, out_hbm, idx_v, rows_v, gs0, gs1, os0, os1):
    n_grp = idx_hbm.shape[0]
    grp_per_w = n_grp // NW
    iters = grp_per_w // K
    wid = lax.axis_index("s") * NC + lax.axis_index("c")
    gbase = wid * grp_per_w
    gsem = [gs0, gs1]
    osem = [os0, os1]

    def stage_idx(slot, g):
        pltpu.sync_copy(idx_hbm.at[pl.ds(gbase + g * K, K)], idx_v.at[slot])

    def fire_gathers(slot):
        for j in range(K):
            pltpu.async_copy(
                table_hbm.at[idx_v.at[slot].at[j]], rows_v.at[slot].at[j],
                gsem[slot])

    def drain_gathers(slot):
        for j in range(K):
            pltpu.make_async_copy(
                table_hbm.at[idx_v.at[slot].at[j]], rows_v.at[slot].at[j],
                gsem[slot]).wait()

    def fire_out(slot, g):
        pltpu.async_copy(
            rows_v.at[slot], out_hbm.at[pl.ds(gbase + g * K, K)], osem[slot])

    def drain_out(slot):
        pltpu.make_async_copy(
            rows_v.at[slot], out_hbm.at[pl.ds(0, K)], osem[slot]).wait()

    def scale_rows(slot):
        def body(i, carry):
            for j in range(K):
                for k in range(D // 16):
                    sl = pl.ds(k * 16, 16)
                    rows_v[slot, j, i, sl] = rows_v[slot, j, i, sl] * SCALE
            return carry
        lax.fori_loop(0, G, body, None)

    stage_idx(0, 0)
    fire_gathers(0)

    def pair(p, carry):
        for b in range(2):
            nb = 1 - b
            g = p * 2 + b
            drain_gathers(b)

            @pl.when(g + 1 < iters)
            def _():
                stage_idx(nb, g + 1)

                @pl.when(g >= 1)
                def _():
                    drain_out(nb)
                fire_gathers(nb)

            scale_rows(b)
            fire_out(b, g)
        return carry

    lax.fori_loop(0, iters // 2, pair, None)
    drain_out((iters - 1) % 2)


def kernel(x, lut):
    B = x.shape[0] * x.shape[1]
    idx = x.reshape(B // G, G)

    mesh = plsc.VectorSubcoreMesh(
        core_axis_name="c", subcore_axis_name="s",
        num_cores=NC, num_subcores=NS,
    )
    emb = pl.kernel(
        _emb_body,
        out_type=jax.ShapeDtypeStruct((B // G, G, D), jnp.float32),
        mesh=mesh,
        scratch_types=[
            pltpu.VMEM((2, K, G), jnp.int32),
            pltpu.VMEM((2, K, G, D), jnp.float32),
            pltpu.SemaphoreType.DMA,
            pltpu.SemaphoreType.DMA,
            pltpu.SemaphoreType.DMA,
            pltpu.SemaphoreType.DMA,
        ],
        compiler_params=pltpu.CompilerParams(use_tc_tiling_on_sc=False),
    )
    out = emb(idx, lut)
    return out.reshape(x.shape[0], x.shape[1], D)
